# baseline (speedup 1.0000x reference)
import functools

import jax
import jax.numpy as jnp
from jax import lax
from jax.experimental import pallas as pl
from jax.experimental.pallas import tpu as pltpu
from jax.experimental.pallas import tpu_sc as plsc

NU = 5000
NN = 10000
D = 128
INCH = 256
E = 320000
NLAYERS = 3

NC, NS = 2, 16
GC = 32
SCE = 128
SG = 2
SGE = SG * SCE
NQ = 40
EPT = NQ * SGE
EPAD = NC * NS * EPT
ST = 624
ST_LAST = NN - (NS - 1) * ST

BLK = 1000
KB = NU // BLK


def _encode_body(x_ref, w_ref, b_ref, o_ref):
    o_ref[...] = jax.nn.sigmoid(
        jnp.dot(x_ref[...], w_ref[0], preferred_element_type=jnp.float32)
        + b_ref[0, 0])


def _encode(x_all, w_enc, b_enc):
    return pl.pallas_call(
        _encode_body,
        grid=(2, KB),
        in_specs=[
            pl.BlockSpec((BLK, INCH), lambda g, i: (g * KB + i, 0)),
            pl.BlockSpec((1, INCH, D), lambda g, i: (g, 0, 0)),
            pl.BlockSpec((1, 1, D), lambda g, i: (g, 0, 0)),
        ],
        out_specs=pl.BlockSpec((BLK, D), lambda g, i: (g * KB + i, 0)),
        out_shape=jax.ShapeDtypeStruct((NN, D), jnp.float32),
    )(x_all, w_enc, b_enc)


def _trans_compute(x, w_ref, b_ref, o_ref):
    q = pl.program_id(0)

    @pl.when((q == 0) | (q == 3))
    def _():
        o_ref[...] = x

    @pl.when((q == 1) | (q == 2))
    def _():
        o_ref[...] = (jnp.dot(x, w_ref[0], preferred_element_type=jnp.float32)
                      + b_ref[0, 0])


def _trans_body1(x_ref, w_ref, b_ref, o_ref):
    _trans_compute(x_ref[...], w_ref, b_ref, o_ref)


def _trans_body2(p_ref, w_ref, b_ref, o_ref):
    _trans_compute(p_ref[0] + p_ref[1], w_ref, b_ref, o_ref)


_W_SPEC = pl.BlockSpec((1, D, D), lambda q, i: (lax.clamp(0, q - 1, 1), 0, 0))
_B_SPEC = pl.BlockSpec((1, 1, D), lambda q, i: (lax.clamp(0, q - 1, 1), 0, 0))
_TABLE_SPEC = pl.BlockSpec((BLK, D), lambda q, i: (q * KB + i, 0))
_TABLE_SHAPE = jax.ShapeDtypeStruct((2 * NN, D), jnp.float32)


def _make_table_from_emb(emb, w4, b4):
    return pl.pallas_call(
        _trans_body1,
        grid=(4, KB),
        in_specs=[
            pl.BlockSpec((BLK, D), lambda q, i: ((q % 2) * KB + i, 0)),
            _W_SPEC, _B_SPEC,
        ],
        out_specs=_TABLE_SPEC,
        out_shape=_TABLE_SHAPE,
    )(emb, w4, b4)


def _make_table_from_partials(p, w4, b4):
    return pl.pallas_call(
        _trans_body2,
        grid=(4, KB),
        in_specs=[
            pl.BlockSpec((2, BLK, D), lambda q, i: (0, (q % 2) * KB + i, 0)),
            _W_SPEC, _B_SPEC,
        ],
        out_specs=_TABLE_SPEC,
        out_shape=_TABLE_SHAPE,
    )(p, w4, b4)


def _final_body(e0_ref, p1_ref, p2_ref, p3_ref, o_ref):
    o_ref[...] = (e0_ref[...]
                  + p1_ref[0] + p1_ref[1]
                  + p2_ref[0] + p2_ref[1]
                  + p3_ref[0] + p3_ref[1]) * 0.25


def _final_mean(emb0, p1, p2, p3):
    pspec = pl.BlockSpec((2, BLK, D), lambda i: (0, i, 0))
    espec = pl.BlockSpec((BLK, D), lambda i: (i, 0))
    return pl.pallas_call(
        _final_body,
        grid=(NN // BLK,),
        in_specs=[espec, pspec, pspec, pspec],
        out_specs=espec,
        out_shape=jax.ShapeDtypeStruct((NN, D), jnp.float32),
    )(emb0, p1, p2, p3)


_MESH = plsc.VectorSubcoreMesh(core_axis_name="c", subcore_axis_name="s",
                               num_cores=NC, num_subcores=NS)


@functools.partial(
    pl.kernel,
    out_type=jax.ShapeDtypeStruct((NC * NN, D), jnp.float32),
    mesh=_MESH,
    scratch_types=[
        pltpu.VMEM((2, SG, SCE), jnp.int32),
        pltpu.VMEM((3 * SGE,), jnp.int32),
        pltpu.VMEM((2 * SGE,), jnp.float32),
        pltpu.VMEM((GC, D), jnp.float32),
        pltpu.VMEM((GC, D), jnp.float32),
        pltpu.VMEM((GC, D), jnp.float32),
        pltpu.VMEM((GC, D), jnp.float32),
        pltpu.VMEM((SCE, D), jnp.float32),
        pltpu.VMEM((SCE, D), jnp.float32),
        pltpu.VMEM_SHARED((NN, D), jnp.float32),
        pltpu.SemaphoreType.DMA,
        pltpu.SemaphoreType.DMA,
        pltpu.SemaphoreType.DMA,
        pltpu.SemaphoreType.DMA,
        pltpu.SemaphoreType.DMA,
        pltpu.SemaphoreType.DMA,
        pltpu.SemaphoreType.DMA,
        pltpu.SemaphoreType.DMA,
    ],
)
def _sc_spmm(table_hbm, rows_hbm, cols_hbm, vals_hbm, zeros_hbm, out_hbm,
             rows_st, cols_st, vals_st, gbuf0, gbuf1, gbuf2, gbuf3,
             sbuf0, sbuf1, acc,
             gsem0, gsem1, gsem2, gsem3, ssem0, ssem1, stc, strv):
    c = lax.axis_index("c")
    s = lax.axis_index("s")
    tid = c * NS + s

    @pl.when(s < NS - 1)
    def _():
        pltpu.sync_copy(zeros_hbm.at[pl.ds(0, ST)],
                        acc.at[pl.ds(pl.multiple_of(s * ST, 8), ST)])

    @pl.when(s == NS - 1)
    def _():
        pltpu.sync_copy(zeros_hbm, acc.at[pl.ds((NS - 1) * ST, ST_LAST)])

    pltpu.sync_copy(rows_hbm.at[tid, 0], rows_st.at[0])
    pltpu.sync_copy(cols_hbm.at[tid, 0], cols_st.at[pl.ds(0, SGE)])
    pltpu.sync_copy(vals_hbm.at[tid, 0], vals_st.at[pl.ds(0, SGE)])
    pltpu.async_copy(cols_hbm.at[tid, 1], cols_st.at[pl.ds(SGE, SGE)], stc)
    plsc.subcore_barrier()

    gbufs = (gbuf0, gbuf1, gbuf2, gbuf3)
    gsems = (gsem0, gsem1, gsem2, gsem3)
    sbufs = (sbuf0, sbuf1)
    ssems = (ssem0, ssem1)

    for k in range(4):
        pltpu.async_copy(table_hbm.at[cols_st.at[pl.ds(k * GC, GC)]],
                         gbufs[k], gsems[k])

    def super_body(q, carry):
        p2 = lax.rem(q, 2)
        p3 = lax.rem(q, 3)
        rvbase = pl.multiple_of(p2 * SGE, 8)
        cbase = pl.multiple_of(p3 * SGE, 8)

        @pl.when(q > 0)
        def _():
            pltpu.make_async_copy(rows_hbm.at[tid, q], rows_st.at[p2],
                                  strv).wait()
            pltpu.make_async_copy(vals_hbm.at[tid, q],
                                  vals_st.at[pl.ds(rvbase, SGE)],
                                  strv).wait()

        @pl.when(q + 2 < NQ)
        def _():
            n3 = lax.rem(q + 2, 3)
            nb = pl.multiple_of(n3 * SGE, 8)
            pltpu.async_copy(cols_hbm.at[tid, q + 2],
                             cols_st.at[pl.ds(nb, SGE)], stc)

        for b in range(2):
            sbuf = sbufs[b]
            ssem = ssems[b]

            @pl.when(q > 0)
            def _():
                pltpu.make_async_copy(sbuf, acc.at[rows_st.at[0, 0]],
                                      ssem).wait()

            if b == 1:
                @pl.when(q + 1 < NQ)
                def _():
                    n2 = lax.rem(q + 1, 2)
                    n3 = lax.rem(q + 1, 3)
                    nb = pl.multiple_of(n3 * SGE, 8)
                    pltpu.make_async_copy(
                        cols_hbm.at[tid, q + 1],
                        cols_st.at[pl.ds(nb, SGE)], stc).wait()
                    pltpu.async_copy(rows_hbm.at[tid, q + 1],
                                     rows_st.at[n2], strv)
                    pltpu.async_copy(vals_hbm.at[tid, q + 1],
                                     vals_st.at[pl.ds(
                                         pl.multiple_of(n2 * SGE, 8), SGE)],
                                     strv)

            for k in range(4):
                gbuf = gbufs[k]
                gsem = gsems[k]
                pltpu.make_async_copy(
                    table_hbm.at[cols_st.at[pl.ds(0, GC)]], gbuf, gsem
                ).wait()

                for g in range(GC // 16):
                    off = rvbase + b * SCE + k * GC + g * 16
                    v16 = vals_st[pl.ds(off, 16)]
                    for l in range(16):
                        vb = jnp.broadcast_to(v16[l], (16,))
                        e = g * 16 + l
                        for h in range(D // 16):
                            sl = pl.ds(h * 16, 16)
                            sbuf[k * GC + e, sl] = gbuf[e, sl] * vb

                if b == 0:
                    pltpu.async_copy(
                        table_hbm.at[
                            cols_st.at[pl.ds(cbase + SCE + k * GC, GC)]],
                        gbuf, gsem)
                else:
                    @pl.when(q + 1 < NQ)
                    def _():
                        n3 = lax.rem(q + 1, 3)
                        nb = pl.multiple_of(n3 * SGE, 8)
                        pltpu.async_copy(
                            table_hbm.at[
                                cols_st.at[pl.ds(nb + k * GC, GC)]],
                            gbuf, gsem)

            pltpu.async_copy(sbuf, acc.at[rows_st.at[p2, b]], ssem,
                             add=True)
        return carry

    lax.fori_loop(0, NQ, super_body, 0)

    for b in range(2):
        pltpu.make_async_copy(sbufs[b], acc.at[rows_st.at[0, 0]],
                              ssems[b]).wait()
    plsc.subcore_barrier()

    @pl.when(s < NS - 1)
    def _():
        off = pl.multiple_of(s * ST, 8)
        pltpu.sync_copy(acc.at[pl.ds(off, ST)],
                        out_hbm.at[pl.ds(c * NN + off, ST)])

    @pl.when(s == NS - 1)
    def _():
        pltpu.sync_copy(acc.at[pl.ds((NS - 1) * ST, ST_LAST)],
                        out_hbm.at[pl.ds(c * NN + (NS - 1) * ST, ST_LAST)])


def kernel(embedding_user, embedding_item, edge_values, user_encode_W,
           user_encode_b, item_encode_W, item_encode_b, u2i_W, u2i_b,
           i2u_W, i2u_b, edge_index):
    f32 = jnp.float32
    rows = edge_index[0]
    cols = edge_index[1]
    cols2 = cols + jnp.where(rows >= NU, NN, 0).astype(jnp.int32)

    pad = EPAD - E
    spread = jnp.arange(pad, dtype=jnp.int32)
    rows_p = jnp.concatenate([rows, spread % NN])
    cols_p = jnp.concatenate([cols2, spread % (2 * NN)])
    vals_p = jnp.concatenate([edge_values.astype(f32),
                              jnp.zeros((pad,), f32)])
    rows_t = rows_p.reshape(NC * NS, NQ, SG, SCE)
    cols_t = cols_p.reshape(NC * NS, NQ, SGE)
    vals_t = vals_p.reshape(NC * NS, NQ, SGE)
    del rows_p, cols_p, vals_p
    zeros_hbm = jnp.zeros((ST_LAST, D), f32)

    x_all = jnp.concatenate([embedding_user, embedding_item], axis=0)
    w_enc = jnp.stack([user_encode_W, item_encode_W])
    b_enc = jnp.stack([user_encode_b, item_encode_b]).reshape(2, 1, D)

    w2 = jnp.stack([i2u_W, u2i_W])
    b2 = jnp.stack([i2u_b, u2i_b]).reshape(2, 1, D)

    emb0 = _encode(x_all, w_enc, b_enc)

    table = _make_table_from_emb(emb0, w2, b2)
    partials = []
    for layer in range(NLAYERS):
        p = _sc_spmm(table, rows_t, cols_t, vals_t, zeros_hbm)
        p = p.reshape(NC, NN, D)
        partials.append(p)
        if layer + 1 < NLAYERS:
            table = _make_table_from_partials(p, w2, b2)

    light = _final_mean(emb0, *partials)
    return (light[:NU], light[NU:])


# speedup vs baseline: 1.2910x; 1.2910x over previous
import functools

import jax
import jax.numpy as jnp
from jax import lax
from jax.experimental import pallas as pl
from jax.experimental.pallas import tpu as pltpu
from jax.experimental.pallas import tpu_sc as plsc

NU = 5000
NN = 10000
D = 128
INCH = 256
E = 320000
NLAYERS = 3

NC, NS = 2, 16
GC = 32
SCE = 128
SG = 2
SGE = SG * SCE
NQ = 40
EPT = NQ * SGE
EPAD = NC * NS * EPT
ST = 624
ST_LAST = NN - (NS - 1) * ST

BLK = 1000
KB = NU // BLK


def _encode_body(x_ref, w_ref, b_ref, o_ref):
    o_ref[...] = jax.nn.sigmoid(
        jnp.dot(x_ref[...], w_ref[0], preferred_element_type=jnp.float32)
        + b_ref[0, 0])


def _encode(x_all, w_enc, b_enc):
    return pl.pallas_call(
        _encode_body,
        grid=(2, KB),
        in_specs=[
            pl.BlockSpec((BLK, INCH), lambda g, i: (g * KB + i, 0)),
            pl.BlockSpec((1, INCH, D), lambda g, i: (g, 0, 0)),
            pl.BlockSpec((1, 1, D), lambda g, i: (g, 0, 0)),
        ],
        out_specs=pl.BlockSpec((BLK, D), lambda g, i: (g * KB + i, 0)),
        out_shape=jax.ShapeDtypeStruct((NN, D), jnp.float32),
    )(x_all, w_enc, b_enc)


def _trans_compute(x, w_ref, b_ref, o_ref):
    q = pl.program_id(0)

    @pl.when((q == 0) | (q == 3))
    def _():
        o_ref[...] = x

    @pl.when((q == 1) | (q == 2))
    def _():
        o_ref[...] = (jnp.dot(x, w_ref[0], preferred_element_type=jnp.float32)
                      + b_ref[0, 0])


def _trans_body1(x_ref, w_ref, b_ref, o_ref):
    _trans_compute(x_ref[...], w_ref, b_ref, o_ref)


def _trans_body2(p_ref, w_ref, b_ref, o_ref):
    _trans_compute(p_ref[0] + p_ref[1], w_ref, b_ref, o_ref)


_W_SPEC = pl.BlockSpec((1, D, D), lambda q, i: (lax.clamp(0, q - 1, 1), 0, 0))
_B_SPEC = pl.BlockSpec((1, 1, D), lambda q, i: (lax.clamp(0, q - 1, 1), 0, 0))
_TABLE_SPEC = pl.BlockSpec((BLK, D), lambda q, i: (q * KB + i, 0))
_TABLE_SHAPE = jax.ShapeDtypeStruct((2 * NN, D), jnp.float32)


def _make_table_from_emb(emb, w4, b4):
    return pl.pallas_call(
        _trans_body1,
        grid=(4, KB),
        in_specs=[
            pl.BlockSpec((BLK, D), lambda q, i: ((q % 2) * KB + i, 0)),
            _W_SPEC, _B_SPEC,
        ],
        out_specs=_TABLE_SPEC,
        out_shape=_TABLE_SHAPE,
    )(emb, w4, b4)


def _make_table_from_partials(p, w4, b4):
    return pl.pallas_call(
        _trans_body2,
        grid=(4, KB),
        in_specs=[
            pl.BlockSpec((2, BLK, D), lambda q, i: (0, (q % 2) * KB + i, 0)),
            _W_SPEC, _B_SPEC,
        ],
        out_specs=_TABLE_SPEC,
        out_shape=_TABLE_SHAPE,
    )(p, w4, b4)


def _final_body(e0_ref, p1_ref, p2_ref, p3_ref, o_ref):
    o_ref[...] = (e0_ref[...]
                  + p1_ref[0] + p1_ref[1]
                  + p2_ref[0] + p2_ref[1]
                  + p3_ref[0] + p3_ref[1]) * 0.25


def _final_mean(emb0, p1, p2, p3):
    pspec = pl.BlockSpec((2, BLK, D), lambda i: (0, i, 0))
    espec = pl.BlockSpec((BLK, D), lambda i: (i, 0))
    return pl.pallas_call(
        _final_body,
        grid=(NN // BLK,),
        in_specs=[espec, pspec, pspec, pspec],
        out_specs=espec,
        out_shape=jax.ShapeDtypeStruct((NN, D), jnp.float32),
    )(emb0, p1, p2, p3)


_MESH = plsc.VectorSubcoreMesh(core_axis_name="c", subcore_axis_name="s",
                               num_cores=NC, num_subcores=NS)


@functools.partial(
    pl.kernel,
    out_type=jax.ShapeDtypeStruct((NC * NN, D), jnp.float32),
    mesh=_MESH,
    scratch_types=[
        pltpu.VMEM((2, SG, SCE), jnp.int32),
        pltpu.VMEM((3 * SGE,), jnp.int32),
        pltpu.VMEM((2 * SGE,), jnp.float32),
        pltpu.VMEM((GC, D), jnp.float32),
        pltpu.VMEM((GC, D), jnp.float32),
        pltpu.VMEM((GC, D), jnp.float32),
        pltpu.VMEM((GC, D), jnp.float32),
        pltpu.VMEM((SCE, D), jnp.float32),
        pltpu.VMEM((SCE, D), jnp.float32),
        pltpu.VMEM_SHARED((NN, D), jnp.float32),
        pltpu.SemaphoreType.DMA,
        pltpu.SemaphoreType.DMA,
        pltpu.SemaphoreType.DMA,
        pltpu.SemaphoreType.DMA,
        pltpu.SemaphoreType.DMA,
        pltpu.SemaphoreType.DMA,
        pltpu.SemaphoreType.DMA,
        pltpu.SemaphoreType.DMA,
    ],
)
def _sc_spmm(table_hbm, rows_hbm, cols_hbm, vals_hbm, zeros_hbm, out_hbm,
             rows_st, cols_st, vals_st, gbuf0, gbuf1, gbuf2, gbuf3,
             sbuf0, sbuf1, acc,
             gsem0, gsem1, gsem2, gsem3, ssem0, ssem1, stc, strv):
    c = lax.axis_index("c")
    s = lax.axis_index("s")
    tid = c * NS + s

    @pl.when(s < NS - 1)
    def _():
        pltpu.sync_copy(zeros_hbm.at[pl.ds(0, ST)],
                        acc.at[pl.ds(pl.multiple_of(s * ST, 8), ST)])

    @pl.when(s == NS - 1)
    def _():
        pltpu.sync_copy(zeros_hbm, acc.at[pl.ds((NS - 1) * ST, ST_LAST)])

    pltpu.sync_copy(rows_hbm.at[tid, 0], rows_st.at[0])
    pltpu.sync_copy(cols_hbm.at[tid, 0], cols_st.at[pl.ds(0, SGE)])
    pltpu.sync_copy(vals_hbm.at[tid, 0], vals_st.at[pl.ds(0, SGE)])
    pltpu.async_copy(cols_hbm.at[tid, 1], cols_st.at[pl.ds(SGE, SGE)], stc)
    plsc.subcore_barrier()

    gbufs = (gbuf0, gbuf1, gbuf2, gbuf3)
    gsems = (gsem0, gsem1, gsem2, gsem3)
    sbufs = (sbuf0, sbuf1)
    ssems = (ssem0, ssem1)

    for k in range(4):
        pltpu.async_copy(table_hbm.at[cols_st.at[pl.ds(k * GC, GC)]],
                         gbufs[k], gsems[k])

    def super_body(q, carry):
        p2 = lax.rem(q, 2)
        p3 = lax.rem(q, 3)
        rvbase = pl.multiple_of(p2 * SGE, 8)
        cbase = pl.multiple_of(p3 * SGE, 8)

        @pl.when(q > 0)
        def _():
            pltpu.make_async_copy(rows_hbm.at[tid, q], rows_st.at[p2],
                                  strv).wait()
            pltpu.make_async_copy(vals_hbm.at[tid, q],
                                  vals_st.at[pl.ds(rvbase, SGE)],
                                  strv).wait()

        @pl.when(q + 2 < NQ)
        def _():
            n3 = lax.rem(q + 2, 3)
            nb = pl.multiple_of(n3 * SGE, 8)
            pltpu.async_copy(cols_hbm.at[tid, q + 2],
                             cols_st.at[pl.ds(nb, SGE)], stc)

        for b in range(2):
            sbuf = sbufs[b]
            ssem = ssems[b]

            @pl.when(q > 0)
            def _():
                pltpu.make_async_copy(sbuf, acc.at[rows_st.at[0, 0]],
                                      ssem).wait()

            if b == 1:
                @pl.when(q + 1 < NQ)
                def _():
                    n2 = lax.rem(q + 1, 2)
                    n3 = lax.rem(q + 1, 3)
                    nb = pl.multiple_of(n3 * SGE, 8)
                    pltpu.make_async_copy(
                        cols_hbm.at[tid, q + 1],
                        cols_st.at[pl.ds(nb, SGE)], stc).wait()
                    pltpu.async_copy(rows_hbm.at[tid, q + 1],
                                     rows_st.at[n2], strv)
                    pltpu.async_copy(vals_hbm.at[tid, q + 1],
                                     vals_st.at[pl.ds(
                                         pl.multiple_of(n2 * SGE, 8), SGE)],
                                     strv)

            for k in range(4):
                gbuf = gbufs[k]
                gsem = gsems[k]
                pltpu.make_async_copy(
                    table_hbm.at[cols_st.at[pl.ds(0, GC)]], gbuf, gsem
                ).wait()

                def scale_body(g, carry3):
                    off = rvbase + b * SCE + k * GC + g * 16
                    v16 = vals_st[pl.ds(off, 16)]
                    for l in range(16):
                        vb = jnp.broadcast_to(v16[l], (16,))
                        e = g * 16 + l
                        for h in range(D // 16):
                            sl = pl.ds(h * 16, 16)
                            sbuf[k * GC + e, sl] = gbuf[e, sl] * vb
                    return carry3

                lax.fori_loop(0, GC // 16, scale_body, 0)

                if b == 0:
                    pltpu.async_copy(
                        table_hbm.at[
                            cols_st.at[pl.ds(cbase + SCE + k * GC, GC)]],
                        gbuf, gsem)
                else:
                    @pl.when(q + 1 < NQ)
                    def _():
                        n3 = lax.rem(q + 1, 3)
                        nb = pl.multiple_of(n3 * SGE, 8)
                        pltpu.async_copy(
                            table_hbm.at[
                                cols_st.at[pl.ds(nb + k * GC, GC)]],
                            gbuf, gsem)

            pltpu.async_copy(sbuf, acc.at[rows_st.at[p2, b]], ssem,
                             add=True)
        return carry

    lax.fori_loop(0, NQ, super_body, 0)

    for b in range(2):
        pltpu.make_async_copy(sbufs[b], acc.at[rows_st.at[0, 0]],
                              ssems[b]).wait()
    plsc.subcore_barrier()

    @pl.when(s < NS - 1)
    def _():
        off = pl.multiple_of(s * ST, 8)
        pltpu.sync_copy(acc.at[pl.ds(off, ST)],
                        out_hbm.at[pl.ds(c * NN + off, ST)])

    @pl.when(s == NS - 1)
    def _():
        pltpu.sync_copy(acc.at[pl.ds((NS - 1) * ST, ST_LAST)],
                        out_hbm.at[pl.ds(c * NN + (NS - 1) * ST, ST_LAST)])


def kernel(embedding_user, embedding_item, edge_values, user_encode_W,
           user_encode_b, item_encode_W, item_encode_b, u2i_W, u2i_b,
           i2u_W, i2u_b, edge_index):
    f32 = jnp.float32
    rows = edge_index[0]
    cols = edge_index[1]
    cols2 = cols + jnp.where(rows >= NU, NN, 0).astype(jnp.int32)

    pad = EPAD - E
    spread = jnp.arange(pad, dtype=jnp.int32)
    rows_p = jnp.concatenate([rows, spread % NN])
    cols_p = jnp.concatenate([cols2, spread % (2 * NN)])
    vals_p = jnp.concatenate([edge_values.astype(f32),
                              jnp.zeros((pad,), f32)])
    rows_t = rows_p.reshape(NC * NS, NQ, SG, SCE)
    cols_t = cols_p.reshape(NC * NS, NQ, SGE)
    vals_t = vals_p.reshape(NC * NS, NQ, SGE)
    del rows_p, cols_p, vals_p
    zeros_hbm = jnp.zeros((ST_LAST, D), f32)

    x_all = jnp.concatenate([embedding_user, embedding_item], axis=0)
    w_enc = jnp.stack([user_encode_W, item_encode_W])
    b_enc = jnp.stack([user_encode_b, item_encode_b]).reshape(2, 1, D)

    w2 = jnp.stack([i2u_W, u2i_W])
    b2 = jnp.stack([i2u_b, u2i_b]).reshape(2, 1, D)

    emb0 = _encode(x_all, w_enc, b_enc)

    table = _make_table_from_emb(emb0, w2, b2)
    partials = []
    for layer in range(NLAYERS):
        p = _sc_spmm(table, rows_t, cols_t, vals_t, zeros_hbm)
        p = p.reshape(NC, NN, D)
        partials.append(p)
        if layer + 1 < NLAYERS:
            table = _make_table_from_partials(p, w2, b2)

    light = _final_mean(emb0, *partials)
    return (light[:NU], light[NU:])
